# baseline (speedup 1.0000x reference)
import functools

import jax
import jax.numpy as jnp
from jax import lax
from jax.experimental import pallas as pl
from jax.experimental.pallas import tpu as pltpu
from jax.experimental.pallas import tpu_sc as plsc

NC = 2
NS = 16
L = 16
NW = NC * NS

B = 16384
D = 128
BPW = B // NW
CHUNK = 16
NCHUNK = BPW // CHUNK
NBUF = 12
AHEAD = NBUF - 2


UNROLL = 4


def _body(s_h, r_h, o_h, ere_h, eim_h, rcat_h, out_h,
          idx_s, idx_r, idx_o,
          sre_b, sim_b, ore_b, oim_b, rcat_b,
          out_v, sem):
    wid = lax.axis_index("s") * NC + lax.axis_index("c")
    base = wid * BPW

    i0 = pltpu.async_copy(s_h.at[pl.ds(base, BPW)], idx_s, sem.at[0])
    i1 = pltpu.async_copy(r_h.at[pl.ds(base, BPW)], idx_r, sem.at[0])
    i2 = pltpu.async_copy(o_h.at[pl.ds(base, BPW)], idx_o, sem.at[0])
    i0.wait(); i1.wait(); i2.wait()

    lane = lax.iota(jnp.int32, L)

    def copies(c, p):
        isl = pl.ds(c * CHUNK, CHUNK)
        return [
            (ere_h.at[idx_s.at[isl]], sre_b.at[p]),
            (eim_h.at[idx_s.at[isl]], sim_b.at[p]),
            (ere_h.at[idx_o.at[isl]], ore_b.at[p]),
            (eim_h.at[idx_o.at[isl]], oim_b.at[p]),
            (rcat_h.at[idx_r.at[isl]], rcat_b.at[p]),
        ]

    def issue(c, p):
        for src, dst in copies(c, p):
            pltpu.async_copy(src, dst, sem.at[p])

    def drain(c, p):
        for src, dst in copies(c, p):
            pltpu.make_async_copy(src, dst, sem.at[p]).wait()

    def compute(c, p):
        def group_body(g, carry):
            def elem_body(i, scores):
                row = g * L + i

                def acc_col(j, rr, ri):
                    dsl = pl.ds(j * L, L)
                    sre = sre_b[p, row, dsl]
                    sim = sim_b[p, row, dsl]
                    orr = ore_b[p, row, dsl]
                    oim = oim_b[p, row, dsl]
                    t1 = rr * orr + ri * oim
                    t2 = rr * oim - ri * orr
                    return sre * t1 + sim * t2

                acc = None
                for jj in range(D // 32):
                    rr0, rr1 = plsc.unpack(
                        plsc.bitcast(rcat_b[p, row, pl.ds(jj * L, L)],
                                     jnp.bfloat16),
                        format=plsc.PackFormat.INTERLEAVED)
                    ri0, ri1 = plsc.unpack(
                        plsc.bitcast(rcat_b[p, row, pl.ds(D // 2 + jj * L, L)],
                                     jnp.bfloat16),
                        format=plsc.PackFormat.INTERLEAVED)
                    for h, rr, ri in ((0, rr0, ri0), (1, rr1, ri1)):
                        a = acc_col(jj * 2 + h, rr, ri)
                        acc = a if acc is None else acc + a
                tot = jnp.sum(acc)
                return jnp.where(lane == i, tot, scores)

            scores = lax.fori_loop(0, L, elem_body,
                                   jnp.zeros((L,), jnp.float32))
            out_v[pl.ds(c * CHUNK + g * L, L)] = scores
            return carry

        lax.fori_loop(0, CHUNK // L, group_body, 0)

    for c in range(AHEAD):
        issue(c, c)

    def chunk_body(c, carry):
        p = lax.rem(c, NBUF)
        drain(c, p)

        @pl.when(c < NCHUNK - AHEAD)
        def _():
            issue(c + AHEAD, lax.rem(c + AHEAD, NBUF))

        compute(c, p)
        return carry

    lax.fori_loop(0, NCHUNK, chunk_body, 0)

    pltpu.sync_copy(out_v, out_h.at[pl.ds(base, BPW)])


def _swizzle_bf16(R):
    n, d = R.shape
    Rb = (R.reshape(n, d // 32, 2, L).swapaxes(2, 3)
          .reshape(n, d // 2, 2).astype(jnp.bfloat16))
    return jax.lax.bitcast_convert_type(Rb, jnp.int32)


@jax.jit
def _score(s, r, o, E_re, E_im, R_re, R_im):
    R_cat = jnp.concatenate(
        [_swizzle_bf16(R_re), _swizzle_bf16(R_im)], axis=1)
    mesh = plsc.VectorSubcoreMesh(core_axis_name="c", subcore_axis_name="s")
    f = functools.partial(
        pl.kernel,
        out_type=jax.ShapeDtypeStruct((B,), jnp.float32),
        mesh=mesh,
        compiler_params=pltpu.CompilerParams(
            needs_layout_passes=False,
            disable_bounds_checks=True,
            disable_semaphore_checks=True,
            skip_device_barrier=True,
        ),
        scratch_types=[
            pltpu.VMEM((BPW,), jnp.int32),
            pltpu.VMEM((BPW,), jnp.int32),
            pltpu.VMEM((BPW,), jnp.int32),
            pltpu.VMEM((NBUF, CHUNK, D), jnp.float32),
            pltpu.VMEM((NBUF, CHUNK, D), jnp.float32),
            pltpu.VMEM((NBUF, CHUNK, D), jnp.float32),
            pltpu.VMEM((NBUF, CHUNK, D), jnp.float32),
            pltpu.VMEM((NBUF, CHUNK, D), jnp.int32),
            pltpu.VMEM((BPW,), jnp.float32),
            pltpu.SemaphoreType.DMA((NBUF,)),
        ],
    )(_body)
    return f(s, r, o, E_re, E_im, R_cat)


def kernel(s, r, o, t, E_re, E_im, R_re, R_im):
    del t
    return _score(s, r, o, E_re, E_im, R_re, R_im)


# speedup vs baseline: 1.0015x; 1.0015x over previous
import functools

import jax
import jax.numpy as jnp
from jax import lax
from jax.experimental import pallas as pl
from jax.experimental.pallas import tpu as pltpu
from jax.experimental.pallas import tpu_sc as plsc

NC = 2
NS = 16
L = 16
NW = NC * NS

B = 16384
D = 128
BPW = B // NW
CHUNK = 32
NCHUNK = BPW // CHUNK
NBUF = 6
AHEAD = NBUF - 1


UNROLL = 4


def _body(s_h, r_h, o_h, ere_h, eim_h, rcat_h, out_h,
          idx_s, idx_r, idx_o,
          sre_b, sim_b, ore_b, oim_b, rcat_b,
          out_v, sem):
    wid = lax.axis_index("s") * NC + lax.axis_index("c")
    base = wid * BPW

    i0 = pltpu.async_copy(s_h.at[pl.ds(base, BPW)], idx_s, sem.at[0])
    i1 = pltpu.async_copy(r_h.at[pl.ds(base, BPW)], idx_r, sem.at[0])
    i2 = pltpu.async_copy(o_h.at[pl.ds(base, BPW)], idx_o, sem.at[0])
    i0.wait(); i1.wait(); i2.wait()

    lane = lax.iota(jnp.int32, L)

    def copies(c, p):
        isl = pl.ds(c * CHUNK, CHUNK)
        return [
            (ere_h.at[idx_s.at[isl]], sre_b.at[p]),
            (eim_h.at[idx_s.at[isl]], sim_b.at[p]),
            (ere_h.at[idx_o.at[isl]], ore_b.at[p]),
            (eim_h.at[idx_o.at[isl]], oim_b.at[p]),
            (rcat_h.at[idx_r.at[isl]], rcat_b.at[p]),
        ]

    def issue(c, p):
        for src, dst in copies(c, p):
            pltpu.async_copy(src, dst, sem.at[p])

    def drain(c, p):
        for src, dst in copies(c, p):
            pltpu.make_async_copy(src, dst, sem.at[p]).wait()

    def compute(c, p):
        def group_body(g, carry):
            def elem_body(i, scores):
                row = g * L + i

                def acc_col(j, rr, ri):
                    dsl = pl.ds(j * L, L)
                    sre = sre_b[p, row, dsl]
                    sim = sim_b[p, row, dsl]
                    orr = ore_b[p, row, dsl]
                    oim = oim_b[p, row, dsl]
                    t1 = rr * orr + ri * oim
                    t2 = rr * oim - ri * orr
                    return sre * t1 + sim * t2

                acc = None
                for jj in range(D // 32):
                    rr0, rr1 = plsc.unpack(
                        plsc.bitcast(rcat_b[p, row, pl.ds(jj * L, L)],
                                     jnp.bfloat16),
                        format=plsc.PackFormat.INTERLEAVED)
                    ri0, ri1 = plsc.unpack(
                        plsc.bitcast(rcat_b[p, row, pl.ds(D // 2 + jj * L, L)],
                                     jnp.bfloat16),
                        format=plsc.PackFormat.INTERLEAVED)
                    for h, rr, ri in ((0, rr0, ri0), (1, rr1, ri1)):
                        a = acc_col(jj * 2 + h, rr, ri)
                        acc = a if acc is None else acc + a
                tot = jnp.sum(acc)
                return jnp.where(lane == i, tot, scores)

            scores = lax.fori_loop(0, L, elem_body,
                                   jnp.zeros((L,), jnp.float32))
            out_v[pl.ds(c * CHUNK + g * L, L)] = scores
            return carry

        lax.fori_loop(0, CHUNK // L, group_body, 0)

    for c in range(AHEAD):
        issue(c, c)

    def chunk_body(c, carry):
        p = lax.rem(c, NBUF)
        drain(c, p)

        @pl.when(c < NCHUNK - AHEAD)
        def _():
            issue(c + AHEAD, lax.rem(c + AHEAD, NBUF))

        compute(c, p)
        return carry

    lax.fori_loop(0, NCHUNK, chunk_body, 0)

    pltpu.sync_copy(out_v, out_h.at[pl.ds(base, BPW)])


def _swizzle_bf16(R):
    n, d = R.shape
    Rb = (R.reshape(n, d // 32, 2, L).swapaxes(2, 3)
          .reshape(n, d // 2, 2).astype(jnp.bfloat16))
    return jax.lax.bitcast_convert_type(Rb, jnp.int32)


@jax.jit
def _score(s, r, o, E_re, E_im, R_re, R_im):
    R_cat = jnp.concatenate(
        [_swizzle_bf16(R_re), _swizzle_bf16(R_im)], axis=1)
    mesh = plsc.VectorSubcoreMesh(core_axis_name="c", subcore_axis_name="s")
    f = functools.partial(
        pl.kernel,
        out_type=jax.ShapeDtypeStruct((B,), jnp.float32),
        mesh=mesh,
        compiler_params=pltpu.CompilerParams(
            needs_layout_passes=False,
            disable_bounds_checks=True,
            disable_semaphore_checks=True,
            skip_device_barrier=True,
        ),
        scratch_types=[
            pltpu.VMEM((BPW,), jnp.int32),
            pltpu.VMEM((BPW,), jnp.int32),
            pltpu.VMEM((BPW,), jnp.int32),
            pltpu.VMEM((NBUF, CHUNK, D), jnp.float32),
            pltpu.VMEM((NBUF, CHUNK, D), jnp.float32),
            pltpu.VMEM((NBUF, CHUNK, D), jnp.float32),
            pltpu.VMEM((NBUF, CHUNK, D), jnp.float32),
            pltpu.VMEM((NBUF, CHUNK, D), jnp.int32),
            pltpu.VMEM((BPW,), jnp.float32),
            pltpu.SemaphoreType.DMA((NBUF,)),
        ],
    )(_body)
    return f(s, r, o, E_re, E_im, R_cat)


def kernel(s, r, o, t, E_re, E_im, R_re, R_im):
    del t
    return _score(s, r, o, E_re, E_im, R_re, R_im)


# speedup vs baseline: 1.0078x; 1.0062x over previous
import functools

import jax
import jax.numpy as jnp
from jax import lax
from jax.experimental import pallas as pl
from jax.experimental.pallas import tpu as pltpu
from jax.experimental.pallas import tpu_sc as plsc

NC = 2
NS = 16
L = 16
NW = NC * NS

B = 16384
D = 128
BPW = B // NW
CHUNK = 32
NCHUNK = BPW // CHUNK
NBUF = 6
AHEAD = NBUF - 2


UNROLL = 4


def _body(s_h, r_h, o_h, ere_h, eim_h, rcat_h, out_h,
          idx_s, idx_r, idx_o,
          sre_b, sim_b, ore_b, oim_b, rcat_b,
          out_v, sem):
    wid = lax.axis_index("s") * NC + lax.axis_index("c")
    base = wid * BPW

    i0 = pltpu.async_copy(s_h.at[pl.ds(base, BPW)], idx_s, sem.at[0])
    i1 = pltpu.async_copy(r_h.at[pl.ds(base, BPW)], idx_r, sem.at[0])
    i2 = pltpu.async_copy(o_h.at[pl.ds(base, BPW)], idx_o, sem.at[0])
    i0.wait(); i1.wait(); i2.wait()

    lane = lax.iota(jnp.int32, L)

    def copies(c, p):
        isl = pl.ds(c * CHUNK, CHUNK)
        return [
            (ere_h.at[idx_s.at[isl]], sre_b.at[p]),
            (eim_h.at[idx_s.at[isl]], sim_b.at[p]),
            (ere_h.at[idx_o.at[isl]], ore_b.at[p]),
            (eim_h.at[idx_o.at[isl]], oim_b.at[p]),
            (rcat_h.at[idx_r.at[isl]], rcat_b.at[p]),
        ]

    def issue(c, p):
        for src, dst in copies(c, p):
            pltpu.async_copy(src, dst, sem.at[p])

    def drain(c, p):
        for src, dst in copies(c, p):
            pltpu.make_async_copy(src, dst, sem.at[p]).wait()

    def compute(c, p):
        def group_body(g, carry):
            def elem_body(i, scores):
                row = g * L + i

                def acc_col(j, rr, ri):
                    dsl = pl.ds(j * L, L)
                    sre = sre_b[p, row, dsl]
                    sim = sim_b[p, row, dsl]
                    orr = ore_b[p, row, dsl]
                    oim = oim_b[p, row, dsl]
                    t1 = rr * orr + ri * oim
                    t2 = rr * oim - ri * orr
                    return sre * t1 + sim * t2

                acc = None
                for jj in range(D // 32):
                    rr0, rr1 = plsc.unpack(
                        plsc.bitcast(rcat_b[p, row, pl.ds(jj * L, L)],
                                     jnp.bfloat16),
                        format=plsc.PackFormat.INTERLEAVED)
                    ri0, ri1 = plsc.unpack(
                        plsc.bitcast(rcat_b[p, row, pl.ds(D // 2 + jj * L, L)],
                                     jnp.bfloat16),
                        format=plsc.PackFormat.INTERLEAVED)
                    for h, rr, ri in ((0, rr0, ri0), (1, rr1, ri1)):
                        a = acc_col(jj * 2 + h, rr, ri)
                        acc = a if acc is None else acc + a
                tot = jnp.sum(acc)
                return jnp.where(lane == i, tot, scores)

            scores = lax.fori_loop(0, L, elem_body,
                                   jnp.zeros((L,), jnp.float32))
            out_v[pl.ds(c * CHUNK + g * L, L)] = scores
            return carry

        lax.fori_loop(0, CHUNK // L, group_body, 0)

    for c in range(AHEAD):
        issue(c, c)

    def chunk_body(c, carry):
        p = lax.rem(c, NBUF)
        drain(c, p)

        @pl.when(c < NCHUNK - AHEAD)
        def _():
            issue(c + AHEAD, lax.rem(c + AHEAD, NBUF))

        compute(c, p)
        return carry

    lax.fori_loop(0, NCHUNK, chunk_body, 0)

    pltpu.sync_copy(out_v, out_h.at[pl.ds(base, BPW)])


def _swizzle_bf16(R):
    n, d = R.shape
    Rb = (R.reshape(n, d // 32, 2, L).swapaxes(2, 3)
          .reshape(n, d // 2, 2).astype(jnp.bfloat16))
    return jax.lax.bitcast_convert_type(Rb, jnp.int32)


@jax.jit
def _score(s, r, o, E_re, E_im, R_re, R_im):
    R_cat = jnp.concatenate(
        [_swizzle_bf16(R_re), _swizzle_bf16(R_im)], axis=1)
    mesh = plsc.VectorSubcoreMesh(core_axis_name="c", subcore_axis_name="s")
    f = functools.partial(
        pl.kernel,
        out_type=jax.ShapeDtypeStruct((B,), jnp.float32),
        mesh=mesh,
        compiler_params=pltpu.CompilerParams(
            needs_layout_passes=False,
            disable_bounds_checks=True,
            disable_semaphore_checks=True,
            skip_device_barrier=True,
        ),
        scratch_types=[
            pltpu.VMEM((BPW,), jnp.int32),
            pltpu.VMEM((BPW,), jnp.int32),
            pltpu.VMEM((BPW,), jnp.int32),
            pltpu.VMEM((NBUF, CHUNK, D), jnp.float32),
            pltpu.VMEM((NBUF, CHUNK, D), jnp.float32),
            pltpu.VMEM((NBUF, CHUNK, D), jnp.float32),
            pltpu.VMEM((NBUF, CHUNK, D), jnp.float32),
            pltpu.VMEM((NBUF, CHUNK, D), jnp.int32),
            pltpu.VMEM((BPW,), jnp.float32),
            pltpu.SemaphoreType.DMA((NBUF,)),
        ],
    )(_body)
    return f(s, r, o, E_re, E_im, R_cat)


def kernel(s, r, o, t, E_re, E_im, R_re, R_im):
    del t
    return _score(s, r, o, E_re, E_im, R_re, R_im)


# speedup vs baseline: 1.0190x; 1.0111x over previous
import functools

import jax
import jax.numpy as jnp
from jax import lax
from jax.experimental import pallas as pl
from jax.experimental.pallas import tpu as pltpu
from jax.experimental.pallas import tpu_sc as plsc

NC = 2
NS = 16
L = 16
NW = NC * NS

B = 16384
D = 128
BPW = B // NW
CHUNK = 32
NCHUNK = BPW // CHUNK
NBUF = 6
AHEAD = NBUF - 2


def _body(s_h, r_h, o_h, ere_h, eim_h, rcat_h, out_h,
          idx_s, idx_r, idx_o,
          sre_b, sim_b, ore_b, oim_b, rcat_b,
          out_v, sem):
    wid = lax.axis_index("s") * NC + lax.axis_index("c")
    base = wid * BPW

    i0 = pltpu.async_copy(s_h.at[pl.ds(base, BPW)], idx_s, sem.at[0])
    i1 = pltpu.async_copy(r_h.at[pl.ds(base, BPW)], idx_r, sem.at[0])
    i2 = pltpu.async_copy(o_h.at[pl.ds(base, BPW)], idx_o, sem.at[0])
    i0.wait(); i1.wait(); i2.wait()

    lane = lax.iota(jnp.int32, L)

    def copies(c, p):
        isl = pl.ds(c * CHUNK, CHUNK)
        return [
            (ere_h.at[idx_s.at[isl]], sre_b.at[p]),
            (eim_h.at[idx_s.at[isl]], sim_b.at[p]),
            (ere_h.at[idx_o.at[isl]], ore_b.at[p]),
            (eim_h.at[idx_o.at[isl]], oim_b.at[p]),
            (rcat_h.at[idx_r.at[isl]], rcat_b.at[p]),
        ]

    def issue(c, p):
        for src, dst in copies(c, p):
            pltpu.async_copy(src, dst, sem.at[p])

    def drain(c, p):
        for src, dst in copies(c, p):
            pltpu.make_async_copy(src, dst, sem.at[p]).wait()

    def compute(c, p):
        def group_body(g, carry):
            def elem_body(i, scores):
                row = g * L + i

                def acc_col(j, rr, ri):
                    dsl = pl.ds(j * L, L)
                    sre = sre_b[p, row, dsl]
                    sim = sim_b[p, row, dsl]
                    orr = ore_b[p, row, dsl]
                    oim = oim_b[p, row, dsl]
                    t1 = rr * orr + ri * oim
                    t2 = rr * oim - ri * orr
                    return sre * t1 + sim * t2

                acc = None
                for jj in range(D // 32):
                    rr0, rr1 = plsc.unpack(
                        plsc.bitcast(rcat_b[p, row, pl.ds(jj * L, L)],
                                     jnp.bfloat16),
                        format=plsc.PackFormat.INTERLEAVED)
                    ri0, ri1 = plsc.unpack(
                        plsc.bitcast(rcat_b[p, row, pl.ds(D // 2 + jj * L, L)],
                                     jnp.bfloat16),
                        format=plsc.PackFormat.INTERLEAVED)
                    for h, rr, ri in ((0, rr0, ri0), (1, rr1, ri1)):
                        a = acc_col(jj * 2 + h, rr, ri)
                        acc = a if acc is None else acc + a
                tot = jnp.sum(acc)
                return jnp.where(lane == i, tot, scores)

            scores = lax.fori_loop(0, L, elem_body,
                                   jnp.zeros((L,), jnp.float32))
            out_v[pl.ds(c * CHUNK + g * L, L)] = scores
            return carry

        lax.fori_loop(0, CHUNK // L, group_body, 0)

    for c in range(AHEAD):
        issue(c, c)

    def chunk_body(c, carry):
        p = lax.rem(c, NBUF)
        drain(c, p)

        @pl.when(c < NCHUNK - AHEAD)
        def _():
            issue(c + AHEAD, lax.rem(c + AHEAD, NBUF))

        compute(c, p)
        return carry

    lax.fori_loop(0, NCHUNK, chunk_body, 0)

    pltpu.sync_copy(out_v, out_h.at[pl.ds(base, BPW)])


def _swizzle_bf16(R):
    n, d = R.shape
    Rb = (R.reshape(n, d // 32, 2, L).swapaxes(2, 3)
          .reshape(n, d // 2, 2).astype(jnp.bfloat16))
    return jax.lax.bitcast_convert_type(Rb, jnp.int32)


@jax.jit
def _score(s, r, o, E_re, E_im, R_re, R_im):
    R_cat = jnp.concatenate(
        [_swizzle_bf16(R_re), _swizzle_bf16(R_im)], axis=1)
    mesh = plsc.VectorSubcoreMesh(core_axis_name="c", subcore_axis_name="s")
    f = functools.partial(
        pl.kernel,
        out_type=jax.ShapeDtypeStruct((B,), jnp.float32),
        mesh=mesh,
        compiler_params=pltpu.CompilerParams(
            needs_layout_passes=False,
            disable_bounds_checks=True,
            disable_semaphore_checks=True,
            skip_device_barrier=True,
        ),
        scratch_types=[
            pltpu.VMEM((BPW,), jnp.int32),
            pltpu.VMEM((BPW,), jnp.int32),
            pltpu.VMEM((BPW,), jnp.int32),
            pltpu.VMEM((NBUF, CHUNK, D), jnp.float32),
            pltpu.VMEM((NBUF, CHUNK, D), jnp.float32),
            pltpu.VMEM((NBUF, CHUNK, D), jnp.float32),
            pltpu.VMEM((NBUF, CHUNK, D), jnp.float32),
            pltpu.VMEM((NBUF, CHUNK, D), jnp.int32),
            pltpu.VMEM((BPW,), jnp.float32),
            pltpu.SemaphoreType.DMA((NBUF,)),
        ],
    )(_body)
    return f(s, r, o, E_re, E_im, R_cat)


def kernel(s, r, o, t, E_re, E_im, R_re, R_im):
    del t
    return _score(s, r, o, E_re, E_im, R_re, R_im)
